# baseline (device time: 59183 ns/iter reference)
import jax
import jax.numpy as jnp
from jax import lax
from jax.experimental import pallas as pl
from jax.experimental.pallas import tpu as pltpu

import os

NCHUNK = int(os.environ.get("NCHUNK", "16"))


def _chunk_rows(half):
    if os.environ.get("CHUNKS", "uniform") == "uniform":
        rows = half // NCHUNK
        return [rows] * NCHUNK
    assert half == 2048
    c = [128] * 15 + [64, 32, 16, 16]
    assert sum(c) == half
    return c
_PROBE = int(os.environ.get("PROBE", "0"))


def kernel(x):
    m, n = x.shape
    half = m // 2
    chunks = _chunk_rows(half)
    nchunk = len(chunks)
    offs = [sum(chunks[:i]) for i in range(nchunk)]

    def body(x_ref, out_ref, xa_oh, xa_h, s1_send, s1_recv, red, s2_recv,
             sems_in_oh, sems_in_h, sems_out, sems_out2,
             sems1_s, sems1_r, sems2_s, sems2_r):
        my_x = lax.axis_index("x")
        my_y = lax.axis_index("y")
        h = my_x ^ my_y
        oh = 1 - h

        in_oh, in_h = [], []
        for c in range(nchunk if _PROBE not in (4, 5, 7) else 0):
            cp = pltpu.make_async_copy(
                x_ref.at[pl.ds(oh * half + offs[c], chunks[c]), :],
                xa_oh.at[pl.ds(offs[c], chunks[c]), :],
                sems_in_oh.at[c],
            )
            cp.start()
            in_oh.append(cp)
            cp = pltpu.make_async_copy(
                x_ref.at[pl.ds(h * half + offs[c], chunks[c]), :],
                xa_h.at[pl.ds(offs[c], chunks[c]), :],
                sems_in_h.at[c],
            )
            cp.start()
            in_h.append(cp)

        barrier_sem = pltpu.get_barrier_semaphore()
        for nbr in ((1 - my_x, my_y), (my_x, 1 - my_y)):
            pl.semaphore_signal(
                barrier_sem, inc=1,
                device_id=nbr, device_id_type=pl.DeviceIdType.MESH,
            )
        pl.semaphore_wait(barrier_sem, 2)

        if _PROBE == 9:
            rx = pltpu.make_async_remote_copy(
                src_ref=s1_send, dst_ref=s1_recv,
                send_sem=sems1_s.at[0], recv_sem=sems1_r.at[0],
                device_id=(1 - my_x, my_y),
                device_id_type=pl.DeviceIdType.MESH,
            )
            rx.start()
            ry = pltpu.make_async_remote_copy(
                src_ref=red, dst_ref=s2_recv,
                send_sem=sems2_s.at[0], recv_sem=sems2_r.at[0],
                device_id=(my_x, 1 - my_y),
                device_id_type=pl.DeviceIdType.MESH,
            )
            ry.start()
            for c in range(nchunk):
                in_oh[c].wait()
                s1_send[pl.ds(offs[c], chunks[c]), :] = (
                    xa_oh[pl.ds(offs[c], chunks[c]), :].astype(jnp.bfloat16)
                )
            for c in range(nchunk):
                in_h[c].wait()
                red[pl.ds(offs[c], chunks[c]), :] = (
                    xa_h[pl.ds(offs[c], chunks[c]), :]
                    + s1_recv[pl.ds(offs[c], chunks[c]), :].astype(jnp.float32)
                ).astype(jnp.bfloat16)
            cp0 = pltpu.make_async_copy(
                red, out_ref.at[pl.ds(0, half), :], sems_out.at[0])
            cp0.start()
            cp1 = pltpu.make_async_copy(
                s2_recv, out_ref.at[pl.ds(half, half), :], sems_out2.at[0])
            cp1.start()
            rx.wait()
            ry.wait()
            cp0.wait()
            cp1.wait()
            return

        if _PROBE == 7:
            rr = []
            for c in range(nchunk):
                r = pltpu.make_async_remote_copy(
                    src_ref=red.at[pl.ds(offs[c], chunks[c]), :],
                    dst_ref=s2_recv.at[pl.ds(offs[c], chunks[c]), :],
                    send_sem=sems2_s.at[c],
                    recv_sem=sems2_r.at[c],
                    device_id=(my_x, 1 - my_y),
                    device_id_type=pl.DeviceIdType.MESH,
                )
                r.start()
                rr.append(r)
            for r in rr:
                r.wait_recv()
            for r in rr:
                r.wait_send()
            return

        if _PROBE in (4, 5):
            r = pltpu.make_async_remote_copy(
                src_ref=s1_send,
                dst_ref=s1_recv,
                send_sem=sems1_s.at[0],
                recv_sem=sems1_r.at[0],
                device_id=(1 - my_x, my_y),
                device_id_type=pl.DeviceIdType.MESH,
            )
            r.start()
            if _PROBE == 5:
                r2 = pltpu.make_async_remote_copy(
                    src_ref=red,
                    dst_ref=s2_recv,
                    send_sem=sems2_s.at[0],
                    recv_sem=sems2_r.at[0],
                    device_id=(my_x, 1 - my_y),
                    device_id_type=pl.DeviceIdType.MESH,
                )
                r2.start()
                r2.wait()
            r.wait()
            return

        rdma1, rdma2, out_cp = [], [], []

        def phase2(c):
            if _PROBE not in (2, 3, 6):
                rdma1[c].wait_recv()
            in_h[c].wait()
            red[pl.ds(offs[c], chunks[c]), :] = (
                xa_h[pl.ds(offs[c], chunks[c]), :]
                + s1_recv[pl.ds(offs[c], chunks[c]), :].astype(jnp.float32)
            ).astype(jnp.bfloat16)
            if _PROBE not in (1, 3):
                r = pltpu.make_async_remote_copy(
                    src_ref=red.at[pl.ds(offs[c], chunks[c]), :],
                    dst_ref=s2_recv.at[pl.ds(offs[c], chunks[c]), :],
                    send_sem=sems2_s.at[c],
                    recv_sem=sems2_r.at[c],
                    device_id=(my_x, 1 - my_y),
                    device_id_type=pl.DeviceIdType.MESH,
                )
                r.start()
                rdma2.append(r)

        for c in range(nchunk):
            in_oh[c].wait()
            s1_send[pl.ds(offs[c], chunks[c]), :] = (
                xa_oh[pl.ds(offs[c], chunks[c]), :].astype(jnp.bfloat16)
            )
            if _PROBE not in (2, 3):
                r = pltpu.make_async_remote_copy(
                    src_ref=s1_send.at[pl.ds(offs[c], chunks[c]), :],
                    dst_ref=s1_recv.at[pl.ds(offs[c], chunks[c]), :],
                    send_sem=sems1_s.at[c],
                    recv_sem=sems1_r.at[c],
                    device_id=(1 - my_x, my_y),
                    device_id_type=pl.DeviceIdType.MESH,
                )
                r.start()
                rdma1.append(r)
        for c in range(nchunk):
            phase2(c)

        cp = pltpu.make_async_copy(
            red, out_ref.at[pl.ds(h * half, half), :], sems_out.at[0])
        cp.start()
        out_cp.append(cp)

        for c, r in enumerate(rdma2):
            r.wait_recv()
            cp = pltpu.make_async_copy(
                s2_recv.at[pl.ds(offs[c], chunks[c]), :],
                out_ref.at[pl.ds(oh * half + offs[c], chunks[c]), :],
                sems_out2.at[c],
            )
            cp.start()
            out_cp.append(cp)
        for cp in out_cp:
            cp.wait()
        if _PROBE == 6:
            for r in rdma1:
                r.wait_recv()
        for r in rdma1:
            r.wait_send()
        for r in rdma2:
            r.wait_send()

    return pl.pallas_call(
        body,
        out_shape=jax.ShapeDtypeStruct((m, n), jnp.bfloat16),
        in_specs=[pl.BlockSpec(memory_space=pl.ANY)],
        out_specs=pl.BlockSpec(memory_space=pl.ANY),
        scratch_shapes=[
            pltpu.VMEM((half, n), jnp.float32),
            pltpu.VMEM((half, n), jnp.float32),
            pltpu.VMEM((half, n), jnp.bfloat16),
            pltpu.VMEM((half, n), jnp.bfloat16),
            pltpu.VMEM((half, n), jnp.bfloat16),
            pltpu.VMEM((half, n), jnp.bfloat16),
            pltpu.SemaphoreType.DMA((nchunk,)),
            pltpu.SemaphoreType.DMA((nchunk,)),
            pltpu.SemaphoreType.DMA((nchunk,)),
            pltpu.SemaphoreType.DMA((nchunk,)),
            pltpu.SemaphoreType.DMA((nchunk,)),
            pltpu.SemaphoreType.DMA((nchunk,)),
            pltpu.SemaphoreType.DMA((nchunk,)),
            pltpu.SemaphoreType.DMA((nchunk,)),
        ],
        compiler_params=pltpu.CompilerParams(collective_id=0),
    )(x)
